# baseline (speedup 1.0000x reference)
import functools

import jax
import jax.numpy as jnp
from jax import lax
from jax.experimental import pallas as pl
from jax.experimental.pallas import tpu as pltpu
from jax.experimental.pallas import tpu_sc as plsc

_MAX_LEN = 8192
_D = 1024
_NC = 2
_NS = 16
_NW = _NC * _NS

_A_T = 160
_CHUNK = 16
_NCHA = _A_T // _CHUNK
_NBUF = 4
_RA = _NW * _A_T

_B_T = 96
_B_C = 16
_NB = _B_T // _B_C
_B_SLOTS = 2
assert _RA + _NW * _B_T == _MAX_LEN


def _sc_copy(table, off_vec):
    mesh = plsc.VectorSubcoreMesh(
        core_axis_name="c", subcore_axis_name="s",
        num_cores=_NC, num_subcores=_NS)

    @functools.partial(
        pl.kernel,
        out_type=jax.ShapeDtypeStruct((_MAX_LEN, _D), jnp.float32),
        mesh=mesh,
        scratch_types=(
            [pltpu.VMEM((16,), jnp.int32),
             pltpu.VMEM((_A_T,), jnp.int32)]
            + [pltpu.VMEM((_CHUNK, _D), jnp.float32) for _ in range(_NBUF)]
            + [pltpu.VMEM_SHARED((_NS * _B_SLOTS * _B_C, _D), jnp.float32)]
            + [pltpu.SemaphoreType.DMA for _ in range(2 * _NBUF + 2 * _B_SLOTS)]
        ),
    )
    def k(table_hbm, off_hbm, out_hbm, off_v, idx_v, *rest):
        bufs = rest[:_NBUF]
        sh = rest[_NBUF]
        gsems = rest[_NBUF + 1:2 * _NBUF + 1]
        wsems = rest[2 * _NBUF + 1:3 * _NBUF + 1]
        bisems = rest[3 * _NBUF + 1:3 * _NBUF + 1 + _B_SLOTS]
        bosems = rest[3 * _NBUF + 1 + _B_SLOTS:]

        cid = lax.axis_index("c")
        sid = lax.axis_index("s")
        wid = cid * _NS + sid

        pltpu.sync_copy(off_hbm, off_v)
        off = off_v[...]
        off_s = pl.multiple_of(off[0], 8)

        abase = wid * _A_T
        lane = lax.iota(jnp.int32, 16)
        for i in range(_A_T // 16):
            v = lane + (abase + 16 * i) + off
            idx_v[pl.ds(16 * i, 16)] = jnp.clip(v, 0, _MAX_LEN - 1)

        def gather(c, s):
            return pltpu.async_copy(
                table_hbm.at[idx_v.at[pl.ds(c * _CHUNK, _CHUNK)]],
                bufs[s], gsems[s])

        def put(c, s):
            return pltpu.async_copy(
                bufs[s], out_hbm.at[pl.ds(abase + c * _CHUNK, _CHUNK)],
                wsems[s])

        bbase = _RA + wid * _B_T
        shbase = sid * _B_SLOTS * _B_C

        def b_in(q):
            s = q % _B_SLOTS
            return pltpu.async_copy(
                table_hbm.at[pl.ds(off_s + bbase + q * _B_C, _B_C)],
                sh.at[pl.ds(shbase + s * _B_C, _B_C)], bisems[s])

        def b_out(q):
            s = q % _B_SLOTS
            return pltpu.async_copy(
                sh.at[pl.ds(shbase + s * _B_C, _B_C)],
                out_hbm.at[pl.ds(bbase + q * _B_C, _B_C)], bosems[s])

        lead = _NBUF - 1
        g = [None] * _NCHA
        w = [None] * _NCHA
        for c in range(min(lead, _NCHA)):
            g[c] = gather(c, c % _NBUF)
        bi = [None] * _NB
        bo = [None] * _NB
        for q in range(_B_SLOTS):
            bi[q] = b_in(q)

        def b_flip(q):
            bi[q].wait()
            bo[q] = b_out(q)

        def b_refill(q):
            bo[q - _B_SLOTS].wait()
            bi[q] = b_in(q)

        b_steps = {0: lambda: b_flip(0), 1: lambda: b_flip(1),
                   2: lambda: b_refill(2), 3: lambda: b_flip(2),
                   4: lambda: b_refill(3), 5: lambda: b_flip(3),
                   6: lambda: b_refill(4), 7: lambda: b_flip(4),
                   8: lambda: b_refill(5), 9: lambda: b_flip(5)}

        unwaited = set()
        for c in range(_NCHA):
            g[c].wait()
            w[c] = put(c, c % _NBUF)
            unwaited.add(c)
            step = b_steps.get(c)
            if step is not None:
                step()
            n = c + lead
            if n < _NCHA:
                if c >= 1:
                    w[c - 1].wait()
                    unwaited.discard(c - 1)
                g[n] = gather(n, n % _NBUF)
        for c in sorted(unwaited):
            w[c].wait()
        bo[_NB - 2].wait()
        bo[_NB - 1].wait()

    return k(table, off_vec)


def kernel(T, pos_emb):
    off_vec = jnp.full((16,), jnp.asarray(T, jnp.int32) - _MAX_LEN,
                       dtype=jnp.int32)
    out = _sc_copy(pos_emb, off_vec)
    return out[None, :, :]
